# Initial kernel scaffold; baseline (speedup 1.0000x reference)
import jax
import jax.numpy as jnp
from jax.experimental import pallas as pl


def kernel(g, h, e, snorm_n, snorm_e, emb, W0, b0, gamma0, beta0, W1, b1, gamma1, beta1, W2, b2, gamma2, beta2, W3, b3, gamma3, beta3, Wr0, br0, Wr1, br1, Wr2, br2):
    raise NotImplementedError("write your pallas kernel here")


# speedup vs baseline: 2.1410x; 2.1410x over previous
import functools

import jax
import jax.numpy as jnp
from jax import lax
from jax.experimental import pallas as pl
from jax.experimental.pallas import tpu as pltpu
from jax.experimental.pallas import tpu_sc as plsc
import numpy as np

N = 10000
NP = 10240
E = 320000
D = 128
NCLS = 8
AVG_D_LOG = float(np.log(32.0))

NTILES = 32
NPH = NP // 2
RPT = NPH // NTILES
ERPT = NP // NTILES
CH = 2048
GK = 128
SUBS = CH // GK
EP = ((E + CH - 1) // CH) * CH
NCHUNK = EP // CH
CB = 2 * GK + 32
FB = 2048
PB = FB + GK + 32
EPF = EP + FB
NGMAX = EPF // GK
ACCR = RPT + 8
SHROWS = 16 * RPT + GK
DUMMY_SH = 16 * RPT
BN = 1024
NB = NP // BN

_mesh = plsc.VectorSubcoreMesh(core_axis_name="c", subcore_axis_name="s")


@functools.partial(
    pl.kernel,
    out_type=jax.ShapeDtypeStruct((NP, D), jnp.float32),
    mesh=_mesh,
    scratch_types=[
        pltpu.VMEM((64,), jnp.int32),
        pltpu.VMEM((64, D), jnp.float32),
        pltpu.SemaphoreType.DMA,
    ],
    compiler_params=pltpu.CompilerParams(needs_layout_passes=False),
)
def _embed(emb_hbm, h_hbm, out_hbm, idx_v, rows_v, sem):
    w = lax.axis_index("s") * 2 + lax.axis_index("c")
    base = w * ERPT
    for g in range(ERPT // 64):
        pltpu.sync_copy(h_hbm.at[pl.ds(base + g * 64, 64)], idx_v)
        pltpu.async_copy(emb_hbm.at[idx_v], rows_v, sem).wait()
        pltpu.sync_copy(rows_v, out_hbm.at[pl.ds(base + g * 64, 64)])


@functools.partial(
    pl.kernel,
    out_type=(
        jax.ShapeDtypeStruct((2 * NTILES * EPF,), jnp.int32),
        jax.ShapeDtypeStruct((2 * NTILES * EPF,), jnp.int32),
        jax.ShapeDtypeStruct((2 * NTILES * 16,), jnp.int32),
    ),
    mesh=_mesh,
    scratch_types=[
        pltpu.VMEM((PB,), jnp.int32),
        pltpu.VMEM((PB,), jnp.int32),
        pltpu.VMEM((PB,), jnp.int32),
        pltpu.VMEM((PB,), jnp.int32),
        pltpu.VMEM((CH,), jnp.int32),
        pltpu.VMEM((CH,), jnp.int32),
        pltpu.VMEM((16,), jnp.int32),
    ],
    compiler_params=pltpu.CompilerParams(needs_layout_passes=False),
)
def _part(src_hbm, dst_hbm, lsrc_hbm, ldl_hbm, lcnt_hbm,
          ps0, pd0, ps1, pd1, esrc, edst, cntb):
    cc = lax.axis_index("c")
    wl = lax.axis_index("s")
    w = wl * 2 + cc
    base0 = w * RPT
    base1 = NPH + w * RPT
    lane = lax.iota(jnp.int32, 16)

    def append(psrc, pdl, cnt, es, ed, m, bs):
        mi = m.astype(jnp.int32)
        pos = plsc.cumsum(mi) - 1
        tgt = jnp.where(m, cnt + pos, PB - 16 + lane)
        plsc.store_scatter(psrc, [tgt], es)
        plsc.store_scatter(pdl, [tgt], ed - bs)
        return cnt + jnp.sum(mi)

    def flush(psrc, pdl, h, cnt, wr):
        full = cnt >= FB
        rbase = pl.multiple_of((h * NTILES + w) * EPF + wr, FB)

        @pl.when(full)
        def _():
            pltpu.sync_copy(psrc.at[pl.ds(0, FB)], lsrc_hbm.at[pl.ds(rbase, FB)])
            pltpu.sync_copy(pdl.at[pl.ds(0, FB)], ldl_hbm.at[pl.ds(rbase, FB)])
            for j in range(GK // 16):
                ssl = pl.ds(FB + j * 16, 16)
                v1 = psrc[ssl]
                v2 = pdl[ssl]
                psrc[pl.ds(j * 16, 16)] = v1
                pdl[pl.ds(j * 16, 16)] = v2
        return (jnp.where(full, cnt - FB, cnt), jnp.where(full, wr + FB, wr))

    def chunk_body(c, st):
        pltpu.sync_copy(src_hbm.at[pl.ds(c * CH, CH)], esrc)
        pltpu.sync_copy(dst_hbm.at[pl.ds(c * CH, CH)], edst)

        def sub_body(si, st):
            c0, w0, c1, w1 = st

            def scan_body(i, st2):
                c0, c1 = st2
                sl = pl.ds(si * GK + i * 16, 16)
                ed = edst[sl]
                es = esrc[sl]
                c0 = append(ps0, pd0, c0, es, ed,
                            (ed >= base0) & (ed < base0 + RPT), base0)
                c1 = append(ps1, pd1, c1, es, ed,
                            (ed >= base1) & (ed < base1 + RPT), base1)
                return c0, c1
            c0, c1 = lax.fori_loop(0, GK // 16, scan_body, (c0, c1))
            c0, w0 = flush(ps0, pd0, 0, c0, w0)
            c1, w1 = flush(ps1, pd1, 1, c1, w1)
            return (c0, w0, c1, w1)
        return lax.fori_loop(0, SUBS, sub_body, st)

    c0, w0, c1, w1 = lax.fori_loop(0, NCHUNK, chunk_body, (0, 0, 0, 0))

    t0 = pl.multiple_of(w * EPF + w0, FB)
    t1 = pl.multiple_of((NTILES + w) * EPF + w1, FB)
    pltpu.sync_copy(ps0.at[pl.ds(0, FB)], lsrc_hbm.at[pl.ds(t0, FB)])
    pltpu.sync_copy(pd0.at[pl.ds(0, FB)], ldl_hbm.at[pl.ds(t0, FB)])
    pltpu.sync_copy(ps1.at[pl.ds(0, FB)], lsrc_hbm.at[pl.ds(t1, FB)])
    pltpu.sync_copy(pd1.at[pl.ds(0, FB)], ldl_hbm.at[pl.ds(t1, FB)])
    cntb[...] = jnp.full((16,), 0, jnp.int32) + (w0 + c0)
    pltpu.sync_copy(cntb, lcnt_hbm.at[pl.ds(pl.multiple_of(w * 16, 16), 16)])
    cntb[...] = jnp.full((16,), 0, jnp.int32) + (w1 + c1)
    pltpu.sync_copy(cntb, lcnt_hbm.at[pl.ds(pl.multiple_of((NTILES + w) * 16, 16), 16)])


def _make_agg(node_base):
  H = node_base // NPH

  @functools.partial(
    pl.kernel,
    out_type=(
        jax.ShapeDtypeStruct((NPH, D), jnp.float32),
        jax.ShapeDtypeStruct((NPH, D), jnp.float32),
        jax.ShapeDtypeStruct((NPH, D), jnp.float32),
        jax.ShapeDtypeStruct((NPH, 16), jnp.float32),
    ),
    mesh=_mesh,
    scratch_types=[
        pltpu.VMEM((GK,), jnp.int32),
        pltpu.VMEM((GK,), jnp.int32),
        pltpu.VMEM((GK,), jnp.int32),
        pltpu.VMEM((16,), jnp.int32),
        pltpu.VMEM((GK, D), jnp.float32),
        pltpu.VMEM((ACCR, D), jnp.float32),
        pltpu.VMEM((ACCR, D), jnp.float32),
        pltpu.VMEM((ACCR, 16), jnp.float32),
        pltpu.VMEM_SHARED((SHROWS, D), jnp.float32),
        pltpu.SemaphoreType.DMA,
    ],
    compiler_params=pltpu.CompilerParams(needs_layout_passes=False),
  )
  def _agg(hf_hbm, lsrc_hbm, ldl_hbm, lcnt_hbm,
           ssum_hbm, smx_hbm, smn_hbm, sdeg_hbm,
           cgath, cdlg, cdl2, cntb, rows, amx, amn, adeg, sh, sem):
    cc = lax.axis_index("c")
    wl = lax.axis_index("s")
    w = wl * 2 + cc
    obase = w * RPT
    shbase = wl * RPT

    zf = jnp.zeros((16,), jnp.float32)
    ninf = jnp.full((16,), -jnp.inf, jnp.float32)
    pinf = jnp.full((16,), jnp.inf, jnp.float32)
    one16 = jnp.full((16,), 1.0, jnp.float32)
    lane = lax.iota(jnp.int32, 16)

    def zrows(i, _):
        for f in range(D // 16):
            rows[i, pl.ds(f * 16, 16)] = zf
        return 0
    lax.fori_loop(0, GK, zrows, 0)

    def zacc(i, _):
        for f in range(D // 16):
            amx[i, pl.ds(f * 16, 16)] = ninf
            amn[i, pl.ds(f * 16, 16)] = pinf
        adeg[i, :] = zf
        return 0
    lax.fori_loop(0, ACCR, zacc, 0)

    pltpu.sync_copy(rows, sh.at[pl.ds(shbase, GK)])
    pltpu.sync_copy(rows.at[pl.ds(0, RPT - GK)], sh.at[pl.ds(shbase + GK, RPT - GK)])

    @pl.when(wl == 0)
    def _():
        pltpu.sync_copy(rows, sh.at[pl.ds(DUMMY_SH, GK)])

    pltpu.sync_copy(lcnt_hbm.at[pl.ds(pl.multiple_of((H * NTILES + w) * 16, 16), 16)], cntb)
    cnt = jnp.max(cntb[...])
    rbase = (H * NTILES + w) * EPF

    def gbody(gi, _):
        @pl.when(gi * GK < cnt)
        def _():
            gof = pl.multiple_of(rbase + gi * GK, GK)
            pltpu.sync_copy(lsrc_hbm.at[pl.ds(gof, GK)], cgath)
            pltpu.sync_copy(ldl_hbm.at[pl.ds(gof, GK)], cdlg)
            for k in range(GK // 16):
                d1 = pl.ds(k * 16, 16)
                valid = (gi * GK + k * 16 + lane) < cnt
                sv = jnp.where(valid, cgath[d1], 0)
                dv = jnp.where(valid, cdlg[d1], RPT)
                cgath[d1] = sv
                cdlg[d1] = dv
                cdl2[d1] = jnp.where(dv >= RPT, DUMMY_SH, dv + shbase)
            pltpu.async_copy(hf_hbm.at[cgath], rows, sem).wait()
            pltpu.sync_copy(rows, sh.at[cdl2], add=True)

            def kbody(k, _):
                eb = k * 16
                for j in range(16):
                    ej = eb + j
                    dlb = plsc.load_gather(cdlg, [jnp.full((16,), ej, jnp.int32)])
                    plsc.addupdate_scatter(adeg, [dlb, lane], one16)
                    for f in range(D // 16):
                        col = lane + (f * 16)
                        r = rows[ej, pl.ds(f * 16, 16)]
                        mxv = plsc.load_gather(amx, [dlb, col])
                        plsc.store_scatter(amx, [dlb, col], jnp.maximum(mxv, r))
                        mnv = plsc.load_gather(amn, [dlb, col])
                        plsc.store_scatter(amn, [dlb, col], jnp.minimum(mnv, r))
                return 0
            lax.fori_loop(0, GK // 16, kbody, 0)
        return 0
    lax.fori_loop(0, NGMAX, gbody, 0)

    pltpu.sync_copy(amx.at[pl.ds(0, RPT)], smx_hbm.at[pl.ds(obase, RPT)])
    pltpu.sync_copy(amn.at[pl.ds(0, RPT)], smn_hbm.at[pl.ds(obase, RPT)])
    pltpu.sync_copy(adeg.at[pl.ds(0, RPT)], sdeg_hbm.at[pl.ds(obase, RPT)])
    pltpu.sync_copy(sh.at[pl.ds(shbase, RPT)], ssum_hbm.at[pl.ds(obase, RPT)])
  return _agg


_agg_lo = _make_agg(0)
_agg_hi = _make_agg(NPH)


def _c1_body(ssum, smx, smn, sdeg, snorm, W, b, hn, stats):
    deg = sdeg[...][:, 0:1]
    degc = jnp.maximum(deg, 1.0)
    mean = ssum[...] / degc
    has = deg > 0.0
    mx = jnp.where(has, smx[...], 0.0)
    mn = jnp.where(has, smn[...], 0.0)
    logd = jnp.log(deg + 1.0)
    amp = logd * (1.0 / AVG_D_LOG)
    att = AVG_D_LOG / jnp.maximum(logd, 1e-6)
    agg = jnp.concatenate([mean, mx, mn], axis=1)
    h1 = (jnp.dot(agg, W[0:3 * D, :], preferred_element_type=jnp.float32)
          + jnp.dot(agg * amp, W[3 * D:6 * D, :], preferred_element_type=jnp.float32)
          + jnp.dot(agg * att, W[6 * D:9 * D, :], preferred_element_type=jnp.float32)
          + b[...])
    h1 = h1 * snorm[...]
    hn[...] = h1
    i = pl.program_id(0)

    @pl.when(i == 0)
    def _():
        stats[...] = jnp.zeros_like(stats)

    rid = i * BN + lax.broadcasted_iota(jnp.int32, (BN, 1), 0)
    valid = rid < N
    hv = jnp.where(valid, h1, 0.0)
    hv2 = jnp.where(valid, h1 * h1, 0.0)
    stats[0:1, :] = stats[0:1, :] + jnp.sum(hv, axis=0, keepdims=True)
    stats[1:2, :] = stats[1:2, :] + jnp.sum(hv2, axis=0, keepdims=True)


_c1 = pl.pallas_call(
    _c1_body,
    grid=(NB,),
    in_specs=[
        pl.BlockSpec((BN, D), lambda i: (i, 0)),
        pl.BlockSpec((BN, D), lambda i: (i, 0)),
        pl.BlockSpec((BN, D), lambda i: (i, 0)),
        pl.BlockSpec((BN, 16), lambda i: (i, 0)),
        pl.BlockSpec((BN, 1), lambda i: (i, 0)),
        pl.BlockSpec((9 * D, D), lambda i: (0, 0)),
        pl.BlockSpec((1, D), lambda i: (0, 0)),
    ],
    out_specs=[
        pl.BlockSpec((BN, D), lambda i: (i, 0)),
        pl.BlockSpec((8, D), lambda i: (0, 0)),
    ],
    out_shape=[
        jax.ShapeDtypeStruct((NP, D), jnp.float32),
        jax.ShapeDtypeStruct((8, D), jnp.float32),
    ],
)


def _c2_body(hn, hf, stats, gamma, beta, out):
    mu = stats[0:1, :] * (1.0 / N)
    ex2 = stats[1:2, :] * (1.0 / N)
    var = ex2 - mu * mu
    inv = lax.rsqrt(var + 1e-5)
    out[...] = hf[...] + jnp.maximum((hn[...] - mu) * inv * gamma[...] + beta[...], 0.0)


_c2 = pl.pallas_call(
    _c2_body,
    grid=(NB,),
    in_specs=[
        pl.BlockSpec((BN, D), lambda i: (i, 0)),
        pl.BlockSpec((BN, D), lambda i: (i, 0)),
        pl.BlockSpec((8, D), lambda i: (0, 0)),
        pl.BlockSpec((1, D), lambda i: (0, 0)),
        pl.BlockSpec((1, D), lambda i: (0, 0)),
    ],
    out_specs=pl.BlockSpec((BN, D), lambda i: (i, 0)),
    out_shape=jax.ShapeDtypeStruct((NP, D), jnp.float32),
)


def _ro_body(hf, w0, b0, w1, b1, w2, b2, out):
    z = jnp.maximum(jnp.dot(hf[...], w0[...], preferred_element_type=jnp.float32) + b0[...], 0.0)
    z = jnp.maximum(jnp.dot(z, w1[...], preferred_element_type=jnp.float32) + b1[...], 0.0)
    out[...] = jnp.dot(z, w2[...], preferred_element_type=jnp.float32) + b2[...]


_ro = pl.pallas_call(
    _ro_body,
    grid=(NB,),
    in_specs=[
        pl.BlockSpec((BN, D), lambda i: (i, 0)),
        pl.BlockSpec((D, D // 2), lambda i: (0, 0)),
        pl.BlockSpec((1, D // 2), lambda i: (0, 0)),
        pl.BlockSpec((D // 2, D // 4), lambda i: (0, 0)),
        pl.BlockSpec((1, D // 4), lambda i: (0, 0)),
        pl.BlockSpec((D // 4, NCLS), lambda i: (0, 0)),
        pl.BlockSpec((1, NCLS), lambda i: (0, 0)),
    ],
    out_specs=pl.BlockSpec((BN, NCLS), lambda i: (i, 0)),
    out_shape=jax.ShapeDtypeStruct((NP, NCLS), jnp.float32),
)


def kernel(g, h, e, snorm_n, snorm_e, emb,
           W0, b0, gamma0, beta0,
           W1, b1, gamma1, beta1,
           W2, b2, gamma2, beta2,
           W3, b3, gamma3, beta3,
           Wr0, br0, Wr1, br1, Wr2, br2):
    src, dst = g[0], g[1]
    hp = jnp.concatenate([h, jnp.zeros((NP - N,), jnp.int32)])
    srcp = jnp.concatenate([src, jnp.zeros((EP - E,), jnp.int32)])
    dstp = jnp.concatenate([dst, jnp.full((EP - E,), 1 << 20, jnp.int32)])
    snp = jnp.concatenate([snorm_n, jnp.ones((NP - N, 1), jnp.float32)], axis=0)

    hf = _embed(emb, hp)
    lsrc, ldl, lcnt = _part(srcp, dstp)
    for (W, b, ga, be) in ((W0, b0, gamma0, beta0), (W1, b1, gamma1, beta1),
                           (W2, b2, gamma2, beta2), (W3, b3, gamma3, beta3)):
        s_lo, mx_lo, mn_lo, dg_lo = _agg_lo(hf, lsrc, ldl, lcnt)
        s_hi, mx_hi, mn_hi, dg_hi = _agg_hi(hf, lsrc, ldl, lcnt)
        ssum = jnp.concatenate([s_lo, s_hi], axis=0)
        smx = jnp.concatenate([mx_lo, mx_hi], axis=0)
        smn = jnp.concatenate([mn_lo, mn_hi], axis=0)
        sdeg = jnp.concatenate([dg_lo, dg_hi], axis=0)
        hn, stats = _c1(ssum, smx, smn, sdeg, snp, W, b.reshape(1, D))
        hf = _c2(hn, hf, stats, ga.reshape(1, D), be.reshape(1, D))
    z = _ro(hf, Wr0, br0.reshape(1, -1), Wr1, br1.reshape(1, -1),
            Wr2, br2.reshape(1, -1))
    return z[:N]


# speedup vs baseline: 3.8811x; 1.8128x over previous
import functools

import jax
import jax.numpy as jnp
from jax import lax
from jax.experimental import pallas as pl
from jax.experimental.pallas import tpu as pltpu
from jax.experimental.pallas import tpu_sc as plsc
import numpy as np

N = 10000
NP = 10240
E = 320000
D = 128
NCLS = 8
AVG_D_LOG = float(np.log(32.0))

NTILES = 32
NPH = NP // 2
RPT = NPH // NTILES
ERPT = NP // NTILES
CH = 2048
GK = 128
SUBS = CH // GK
EP = ((E + CH - 1) // CH) * CH
NCHUNK = EP // CH
CB = 2 * GK + 32
FB = 2048
PB = FB + GK + 32
EPF = EP + FB
NGMAX = EPF // GK
ACCR = RPT + 8
SHROWS = 16 * RPT + GK
DUMMY_SH = 16 * RPT
BN = 1024
NB = NP // BN

_mesh = plsc.VectorSubcoreMesh(core_axis_name="c", subcore_axis_name="s")


@functools.partial(
    pl.kernel,
    out_type=jax.ShapeDtypeStruct((NP, D), jnp.float32),
    mesh=_mesh,
    scratch_types=[
        pltpu.VMEM((64,), jnp.int32),
        pltpu.VMEM((64, D), jnp.float32),
        pltpu.SemaphoreType.DMA,
    ],
    compiler_params=pltpu.CompilerParams(needs_layout_passes=False),
)
def _embed(emb_hbm, h_hbm, out_hbm, idx_v, rows_v, sem):
    w = lax.axis_index("s") * 2 + lax.axis_index("c")
    base = w * ERPT
    for g in range(ERPT // 64):
        pltpu.sync_copy(h_hbm.at[pl.ds(base + g * 64, 64)], idx_v)
        pltpu.async_copy(emb_hbm.at[idx_v], rows_v, sem).wait()
        pltpu.sync_copy(rows_v, out_hbm.at[pl.ds(base + g * 64, 64)])


@functools.partial(
    pl.kernel,
    out_type=(
        jax.ShapeDtypeStruct((2 * NTILES * EPF,), jnp.int32),
        jax.ShapeDtypeStruct((2 * NTILES * EPF,), jnp.int32),
        jax.ShapeDtypeStruct((2 * NTILES * 16,), jnp.int32),
    ),
    mesh=_mesh,
    scratch_types=[
        pltpu.VMEM((PB,), jnp.int32),
        pltpu.VMEM((PB,), jnp.int32),
        pltpu.VMEM((PB,), jnp.int32),
        pltpu.VMEM((PB,), jnp.int32),
        pltpu.VMEM((CH,), jnp.int32),
        pltpu.VMEM((CH,), jnp.int32),
        pltpu.VMEM((16,), jnp.int32),
    ],
    compiler_params=pltpu.CompilerParams(needs_layout_passes=False),
)
def _part(src_hbm, dst_hbm, lsrc_hbm, ldl_hbm, lcnt_hbm,
          ps0, pd0, ps1, pd1, esrc, edst, cntb):
    cc = lax.axis_index("c")
    wl = lax.axis_index("s")
    w = wl * 2 + cc
    base0 = w * RPT
    base1 = NPH + w * RPT
    lane = lax.iota(jnp.int32, 16)

    def append(psrc, pdl, cnt, es, ed, m, bs):
        mi = m.astype(jnp.int32)
        pos = plsc.cumsum(mi) - 1
        tgt = jnp.where(m, cnt + pos, PB - 16 + lane)
        plsc.store_scatter(psrc, [tgt], es)
        plsc.store_scatter(pdl, [tgt], ed - bs)
        return cnt + jnp.sum(mi)

    def flush(psrc, pdl, h, cnt, wr):
        full = cnt >= FB
        rbase = pl.multiple_of((h * NTILES + w) * EPF + wr, FB)

        @pl.when(full)
        def _():
            pltpu.sync_copy(psrc.at[pl.ds(0, FB)], lsrc_hbm.at[pl.ds(rbase, FB)])
            pltpu.sync_copy(pdl.at[pl.ds(0, FB)], ldl_hbm.at[pl.ds(rbase, FB)])
            for j in range(GK // 16):
                ssl = pl.ds(FB + j * 16, 16)
                v1 = psrc[ssl]
                v2 = pdl[ssl]
                psrc[pl.ds(j * 16, 16)] = v1
                pdl[pl.ds(j * 16, 16)] = v2
        return (jnp.where(full, cnt - FB, cnt), jnp.where(full, wr + FB, wr))

    def chunk_body(c, st):
        pltpu.sync_copy(src_hbm.at[pl.ds(c * CH, CH)], esrc)
        pltpu.sync_copy(dst_hbm.at[pl.ds(c * CH, CH)], edst)

        def sub_body(si, st):
            c0, w0, c1, w1 = st

            def scan_body(i, st2):
                c0, c1 = st2
                sl = pl.ds(si * GK + i * 16, 16)
                ed = edst[sl]
                es = esrc[sl]
                c0 = append(ps0, pd0, c0, es, ed,
                            (ed >= base0) & (ed < base0 + RPT), base0)
                c1 = append(ps1, pd1, c1, es, ed,
                            (ed >= base1) & (ed < base1 + RPT), base1)
                return c0, c1
            c0, c1 = lax.fori_loop(0, GK // 16, scan_body, (c0, c1))
            c0, w0 = flush(ps0, pd0, 0, c0, w0)
            c1, w1 = flush(ps1, pd1, 1, c1, w1)
            return (c0, w0, c1, w1)
        return lax.fori_loop(0, SUBS, sub_body, st)

    c0, w0, c1, w1 = lax.fori_loop(0, NCHUNK, chunk_body, (0, 0, 0, 0))

    t0 = pl.multiple_of(w * EPF + w0, FB)
    t1 = pl.multiple_of((NTILES + w) * EPF + w1, FB)
    pltpu.sync_copy(ps0.at[pl.ds(0, FB)], lsrc_hbm.at[pl.ds(t0, FB)])
    pltpu.sync_copy(pd0.at[pl.ds(0, FB)], ldl_hbm.at[pl.ds(t0, FB)])
    pltpu.sync_copy(ps1.at[pl.ds(0, FB)], lsrc_hbm.at[pl.ds(t1, FB)])
    pltpu.sync_copy(pd1.at[pl.ds(0, FB)], ldl_hbm.at[pl.ds(t1, FB)])
    cntb[...] = jnp.full((16,), 0, jnp.int32) + (w0 + c0)
    pltpu.sync_copy(cntb, lcnt_hbm.at[pl.ds(pl.multiple_of(w * 16, 16), 16)])
    cntb[...] = jnp.full((16,), 0, jnp.int32) + (w1 + c1)
    pltpu.sync_copy(cntb, lcnt_hbm.at[pl.ds(pl.multiple_of((NTILES + w) * 16, 16), 16)])


def _make_agg(node_base):
  H = node_base // NPH

  @functools.partial(
    pl.kernel,
    out_type=(
        jax.ShapeDtypeStruct((NPH, D), jnp.float32),
        jax.ShapeDtypeStruct((NPH, D), jnp.float32),
        jax.ShapeDtypeStruct((NPH, D), jnp.float32),
        jax.ShapeDtypeStruct((NPH, 16), jnp.float32),
    ),
    mesh=_mesh,
    scratch_types=[
        pltpu.VMEM((GK,), jnp.int32),
        pltpu.VMEM((GK,), jnp.int32),
        pltpu.VMEM((GK,), jnp.int32),
        pltpu.VMEM((GK,), jnp.int32),
        pltpu.VMEM((GK,), jnp.int32),
        pltpu.VMEM((GK,), jnp.int32),
        pltpu.VMEM((16,), jnp.int32),
        pltpu.VMEM((GK, D), jnp.float32),
        pltpu.VMEM((GK, D), jnp.float32),
        pltpu.VMEM((ACCR, D), jnp.float32),
        pltpu.VMEM((ACCR, D), jnp.float32),
        pltpu.VMEM((ACCR, 16), jnp.float32),
        pltpu.VMEM_SHARED((SHROWS, D), jnp.float32),
        pltpu.SemaphoreType.DMA,
        pltpu.SemaphoreType.DMA,
    ],
    compiler_params=pltpu.CompilerParams(needs_layout_passes=False),
  )
  def _agg(hf_hbm, lsrc_hbm, ldl_hbm, lcnt_hbm,
           ssum_hbm, smx_hbm, smn_hbm, sdeg_hbm,
           cgath, cdlg, cdl2, cgath2, cdlg2, cdl22, cntb,
           rows, rows2, amx, amn, adeg, sh, sem, sem2):
    cc = lax.axis_index("c")
    wl = lax.axis_index("s")
    w = wl * 2 + cc
    obase = w * RPT
    shbase = wl * RPT

    zf = jnp.zeros((16,), jnp.float32)
    ninf = jnp.full((16,), -jnp.inf, jnp.float32)
    pinf = jnp.full((16,), jnp.inf, jnp.float32)
    one16 = jnp.full((16,), 1.0, jnp.float32)
    lane = lax.iota(jnp.int32, 16)

    def zrows(i, _):
        for f in range(D // 16):
            rows[i, pl.ds(f * 16, 16)] = zf
        return 0
    lax.fori_loop(0, GK, zrows, 0)

    def zacc(i, _):
        for f in range(D // 16):
            amx[i, pl.ds(f * 16, 16)] = ninf
            amn[i, pl.ds(f * 16, 16)] = pinf
        adeg[i, :] = zf
        return 0
    lax.fori_loop(0, ACCR, zacc, 0)

    pltpu.sync_copy(rows, sh.at[pl.ds(shbase, GK)])
    pltpu.sync_copy(rows.at[pl.ds(0, RPT - GK)], sh.at[pl.ds(shbase + GK, RPT - GK)])

    @pl.when(wl == 0)
    def _():
        pltpu.sync_copy(rows, sh.at[pl.ds(DUMMY_SH, GK)])

    pltpu.sync_copy(lcnt_hbm.at[pl.ds(pl.multiple_of((H * NTILES + w) * 16, 16), 16)], cntb)
    cnt = jnp.max(cntb[...])
    rbase = (H * NTILES + w) * EPF

    bufs = ((cgath, cdlg, cdl2, rows, sem), (cgath2, cdlg2, cdl22, rows2, sem2))
    nf = D // 16

    def load_group(gi, b):
        cg, cd, c2, rw, sm = bufs[b]
        gof = pl.multiple_of(rbase + gi * GK, GK)
        pltpu.sync_copy(lsrc_hbm.at[pl.ds(gof, GK)], cg)
        pltpu.sync_copy(ldl_hbm.at[pl.ds(gof, GK)], cd)
        for k in range(GK // 16):
            d1 = pl.ds(k * 16, 16)
            valid = (gi * GK + k * 16 + lane) < cnt
            sv = jnp.where(valid, cg[d1], 0)
            dv = jnp.where(valid, cd[d1], RPT)
            cg[d1] = sv
            cd[d1] = dv
            c2[d1] = jnp.where(dv >= RPT, DUMMY_SH, dv + shbase)
        pltpu.async_copy(hf_hbm.at[cg], rw, sm)

    def process_group(b):
        cg, cd, c2, rw, sm = bufs[b]
        pltpu.make_async_copy(hf_hbm.at[cg], rw, sm).wait()
        pltpu.sync_copy(rw, sh.at[c2], add=True)

        def kbody(k, _):
            eb = k * 16
            dlv = cd[pl.ds(eb, 16)]
            for j in range(16):
                ej = eb + j
                dlb = dlv[jnp.full((16,), j, jnp.int32)]
                plsc.addupdate_scatter(adeg, [dlb, lane], one16)
                cols = [lane + (f * 16) for f in range(nf)]
                rs = [rw[ej, pl.ds(f * 16, 16)] for f in range(nf)]
                mxs = [plsc.load_gather(amx, [dlb, cols[f]]) for f in range(nf)]
                for f in range(nf):
                    plsc.store_scatter(amx, [dlb, cols[f]],
                                       jnp.maximum(mxs[f], rs[f]))
                mns = [plsc.load_gather(amn, [dlb, cols[f]]) for f in range(nf)]
                for f in range(nf):
                    plsc.store_scatter(amn, [dlb, cols[f]],
                                       jnp.minimum(mns[f], rs[f]))
            return 0
        lax.fori_loop(0, GK // 16, kbody, 0)

    @pl.when(0 < cnt)
    def _():
        load_group(0, 0)

    def gouter(go, _):
        for par in range(2):
            gi2 = go * 2 + par

            @pl.when(gi2 * GK < cnt)
            def _(gi2=gi2, par=par):
                @pl.when((gi2 + 1) * GK < cnt)
                def _():
                    load_group(gi2 + 1, 1 - par)
                process_group(par)
        return 0
    lax.fori_loop(0, NGMAX // 2 + 1, gouter, 0)

    pltpu.sync_copy(amx.at[pl.ds(0, RPT)], smx_hbm.at[pl.ds(obase, RPT)])
    pltpu.sync_copy(amn.at[pl.ds(0, RPT)], smn_hbm.at[pl.ds(obase, RPT)])
    pltpu.sync_copy(adeg.at[pl.ds(0, RPT)], sdeg_hbm.at[pl.ds(obase, RPT)])
    pltpu.sync_copy(sh.at[pl.ds(shbase, RPT)], ssum_hbm.at[pl.ds(obase, RPT)])
  return _agg


_agg_lo = _make_agg(0)
_agg_hi = _make_agg(NPH)


def _c1_body(ssum, smx, smn, sdeg, snorm, W, b, hn, stats):
    deg = sdeg[...][:, 0:1]
    degc = jnp.maximum(deg, 1.0)
    mean = ssum[...] / degc
    has = deg > 0.0
    mx = jnp.where(has, smx[...], 0.0)
    mn = jnp.where(has, smn[...], 0.0)
    logd = jnp.log(deg + 1.0)
    amp = logd * (1.0 / AVG_D_LOG)
    att = AVG_D_LOG / jnp.maximum(logd, 1e-6)
    agg = jnp.concatenate([mean, mx, mn], axis=1)
    h1 = (jnp.dot(agg, W[0:3 * D, :], preferred_element_type=jnp.float32)
          + jnp.dot(agg * amp, W[3 * D:6 * D, :], preferred_element_type=jnp.float32)
          + jnp.dot(agg * att, W[6 * D:9 * D, :], preferred_element_type=jnp.float32)
          + b[...])
    h1 = h1 * snorm[...]
    hn[...] = h1
    i = pl.program_id(0)

    @pl.when(i == 0)
    def _():
        stats[...] = jnp.zeros_like(stats)

    rid = i * BN + lax.broadcasted_iota(jnp.int32, (BN, 1), 0)
    valid = rid < N
    hv = jnp.where(valid, h1, 0.0)
    hv2 = jnp.where(valid, h1 * h1, 0.0)
    stats[0:1, :] = stats[0:1, :] + jnp.sum(hv, axis=0, keepdims=True)
    stats[1:2, :] = stats[1:2, :] + jnp.sum(hv2, axis=0, keepdims=True)


_c1 = pl.pallas_call(
    _c1_body,
    grid=(NB,),
    in_specs=[
        pl.BlockSpec((BN, D), lambda i: (i, 0)),
        pl.BlockSpec((BN, D), lambda i: (i, 0)),
        pl.BlockSpec((BN, D), lambda i: (i, 0)),
        pl.BlockSpec((BN, 16), lambda i: (i, 0)),
        pl.BlockSpec((BN, 1), lambda i: (i, 0)),
        pl.BlockSpec((9 * D, D), lambda i: (0, 0)),
        pl.BlockSpec((1, D), lambda i: (0, 0)),
    ],
    out_specs=[
        pl.BlockSpec((BN, D), lambda i: (i, 0)),
        pl.BlockSpec((8, D), lambda i: (0, 0)),
    ],
    out_shape=[
        jax.ShapeDtypeStruct((NP, D), jnp.float32),
        jax.ShapeDtypeStruct((8, D), jnp.float32),
    ],
)


def _c2_body(hn, hf, stats, gamma, beta, out):
    mu = stats[0:1, :] * (1.0 / N)
    ex2 = stats[1:2, :] * (1.0 / N)
    var = ex2 - mu * mu
    inv = lax.rsqrt(var + 1e-5)
    out[...] = hf[...] + jnp.maximum((hn[...] - mu) * inv * gamma[...] + beta[...], 0.0)


_c2 = pl.pallas_call(
    _c2_body,
    grid=(NB,),
    in_specs=[
        pl.BlockSpec((BN, D), lambda i: (i, 0)),
        pl.BlockSpec((BN, D), lambda i: (i, 0)),
        pl.BlockSpec((8, D), lambda i: (0, 0)),
        pl.BlockSpec((1, D), lambda i: (0, 0)),
        pl.BlockSpec((1, D), lambda i: (0, 0)),
    ],
    out_specs=pl.BlockSpec((BN, D), lambda i: (i, 0)),
    out_shape=jax.ShapeDtypeStruct((NP, D), jnp.float32),
)


def _ro_body(hf, w0, b0, w1, b1, w2, b2, out):
    z = jnp.maximum(jnp.dot(hf[...], w0[...], preferred_element_type=jnp.float32) + b0[...], 0.0)
    z = jnp.maximum(jnp.dot(z, w1[...], preferred_element_type=jnp.float32) + b1[...], 0.0)
    out[...] = jnp.dot(z, w2[...], preferred_element_type=jnp.float32) + b2[...]


_ro = pl.pallas_call(
    _ro_body,
    grid=(NB,),
    in_specs=[
        pl.BlockSpec((BN, D), lambda i: (i, 0)),
        pl.BlockSpec((D, D // 2), lambda i: (0, 0)),
        pl.BlockSpec((1, D // 2), lambda i: (0, 0)),
        pl.BlockSpec((D // 2, D // 4), lambda i: (0, 0)),
        pl.BlockSpec((1, D // 4), lambda i: (0, 0)),
        pl.BlockSpec((D // 4, NCLS), lambda i: (0, 0)),
        pl.BlockSpec((1, NCLS), lambda i: (0, 0)),
    ],
    out_specs=pl.BlockSpec((BN, NCLS), lambda i: (i, 0)),
    out_shape=jax.ShapeDtypeStruct((NP, NCLS), jnp.float32),
)


def kernel(g, h, e, snorm_n, snorm_e, emb,
           W0, b0, gamma0, beta0,
           W1, b1, gamma1, beta1,
           W2, b2, gamma2, beta2,
           W3, b3, gamma3, beta3,
           Wr0, br0, Wr1, br1, Wr2, br2):
    src, dst = g[0], g[1]
    hp = jnp.concatenate([h, jnp.zeros((NP - N,), jnp.int32)])
    srcp = jnp.concatenate([src, jnp.zeros((EP - E,), jnp.int32)])
    dstp = jnp.concatenate([dst, jnp.full((EP - E,), 1 << 20, jnp.int32)])
    snp = jnp.concatenate([snorm_n, jnp.ones((NP - N, 1), jnp.float32)], axis=0)

    hf = _embed(emb, hp)
    lsrc, ldl, lcnt = _part(srcp, dstp)
    for (W, b, ga, be) in ((W0, b0, gamma0, beta0), (W1, b1, gamma1, beta1),
                           (W2, b2, gamma2, beta2), (W3, b3, gamma3, beta3)):
        s_lo, mx_lo, mn_lo, dg_lo = _agg_lo(hf, lsrc, ldl, lcnt)
        s_hi, mx_hi, mn_hi, dg_hi = _agg_hi(hf, lsrc, ldl, lcnt)
        ssum = jnp.concatenate([s_lo, s_hi], axis=0)
        smx = jnp.concatenate([mx_lo, mx_hi], axis=0)
        smn = jnp.concatenate([mn_lo, mn_hi], axis=0)
        sdeg = jnp.concatenate([dg_lo, dg_hi], axis=0)
        hn, stats = _c1(ssum, smx, smn, sdeg, snp, W, b.reshape(1, D))
        hf = _c2(hn, hf, stats, ga.reshape(1, D), be.reshape(1, D))
    z = _ro(hf, Wr0, br0.reshape(1, -1), Wr1, br1.reshape(1, -1),
            Wr2, br2.reshape(1, -1))
    return z[:N]


# speedup vs baseline: 4.1426x; 1.0674x over previous
import functools

import jax
import jax.numpy as jnp
from jax import lax
from jax.experimental import pallas as pl
from jax.experimental.pallas import tpu as pltpu
from jax.experimental.pallas import tpu_sc as plsc
import numpy as np

N = 10000
NP = 10240
E = 320000
D = 128
NCLS = 8
AVG_D_LOG = float(np.log(32.0))

NTILES = 32
NPH = NP // 2
RPT = NPH // NTILES
ERPT = NP // NTILES
CH = 2048
GK = 128
SUBS = CH // GK
EP = ((E + CH - 1) // CH) * CH
NCHUNK = EP // CH
CB = 2 * GK + 32
FB = 2048
PB = FB + GK + 32
EPF = EP + FB
NGMAX = EPF // GK
ACCR = RPT + 8
SHROWS = 16 * RPT + GK
DUMMY_SH = 16 * RPT
BN = 1024
NB = NP // BN

_mesh = plsc.VectorSubcoreMesh(core_axis_name="c", subcore_axis_name="s")


@functools.partial(
    pl.kernel,
    out_type=jax.ShapeDtypeStruct((NP, D), jnp.float32),
    mesh=_mesh,
    scratch_types=[
        pltpu.VMEM((64,), jnp.int32),
        pltpu.VMEM((64, D), jnp.float32),
        pltpu.SemaphoreType.DMA,
    ],
    compiler_params=pltpu.CompilerParams(needs_layout_passes=False),
)
def _embed(emb_hbm, h_hbm, out_hbm, idx_v, rows_v, sem):
    w = lax.axis_index("s") * 2 + lax.axis_index("c")
    base = w * ERPT
    for g in range(ERPT // 64):
        pltpu.sync_copy(h_hbm.at[pl.ds(base + g * 64, 64)], idx_v)
        pltpu.async_copy(emb_hbm.at[idx_v], rows_v, sem).wait()
        pltpu.sync_copy(rows_v, out_hbm.at[pl.ds(base + g * 64, 64)])


@functools.partial(
    pl.kernel,
    out_type=(
        jax.ShapeDtypeStruct((2 * NTILES * EPF,), jnp.int32),
        jax.ShapeDtypeStruct((2 * NTILES * EPF,), jnp.int32),
        jax.ShapeDtypeStruct((2 * NTILES * 16,), jnp.int32),
    ),
    mesh=_mesh,
    scratch_types=[
        pltpu.VMEM((PB,), jnp.int32),
        pltpu.VMEM((PB,), jnp.int32),
        pltpu.VMEM((PB,), jnp.int32),
        pltpu.VMEM((PB,), jnp.int32),
        pltpu.VMEM((CH,), jnp.int32),
        pltpu.VMEM((CH,), jnp.int32),
        pltpu.VMEM((CH,), jnp.int32),
        pltpu.VMEM((CH,), jnp.int32),
        pltpu.VMEM((16,), jnp.int32),
        pltpu.SemaphoreType.DMA,
        pltpu.SemaphoreType.DMA,
    ],
    compiler_params=pltpu.CompilerParams(needs_layout_passes=False),
)
def _part(src_hbm, dst_hbm, lsrc_hbm, ldl_hbm, lcnt_hbm,
          ps0, pd0, ps1, pd1, esrcA, edstA, esrcB, edstB, cntb, semA, semB):
    cc = lax.axis_index("c")
    wl = lax.axis_index("s")
    w = wl * 2 + cc
    base0 = w * RPT
    base1 = NPH + w * RPT
    lane = lax.iota(jnp.int32, 16)

    def append(psrc, pdl, cnt, es, ed, m, bs):
        mi = m.astype(jnp.int32)
        pos = plsc.cumsum(mi) - 1
        tgt = jnp.where(m, cnt + pos, PB - 16 + lane)
        plsc.store_scatter(psrc, [tgt], es)
        plsc.store_scatter(pdl, [tgt], ed - bs)
        return cnt + jnp.sum(mi)

    def flush(psrc, pdl, h, cnt, wr):
        full = cnt >= FB
        rbase = pl.multiple_of((h * NTILES + w) * EPF + wr, FB)

        @pl.when(full)
        def _():
            pltpu.sync_copy(psrc.at[pl.ds(0, FB)], lsrc_hbm.at[pl.ds(rbase, FB)])
            pltpu.sync_copy(pdl.at[pl.ds(0, FB)], ldl_hbm.at[pl.ds(rbase, FB)])
            for j in range(GK // 16):
                ssl = pl.ds(FB + j * 16, 16)
                v1 = psrc[ssl]
                v2 = pdl[ssl]
                psrc[pl.ds(j * 16, 16)] = v1
                pdl[pl.ds(j * 16, 16)] = v2
        return (jnp.where(full, cnt - FB, cnt), jnp.where(full, wr + FB, wr))

    stg = ((esrcA, edstA, semA), (esrcB, edstB, semB))

    def stage_start(c, b):
        es, ed, sm = stg[b]
        pltpu.async_copy(src_hbm.at[pl.ds(c * CH, CH)], es, sm)
        pltpu.async_copy(dst_hbm.at[pl.ds(c * CH, CH)], ed, sm)

    def stage_wait(c, b):
        es, ed, sm = stg[b]
        pltpu.make_async_copy(src_hbm.at[pl.ds(c * CH, CH)], es, sm).wait()
        pltpu.make_async_copy(dst_hbm.at[pl.ds(c * CH, CH)], ed, sm).wait()

    def scan_chunk(b, st):
        esrc, edst = stg[b][0], stg[b][1]

        def sub_body(si, st):
            c0, w0, c1, w1 = st

            def scan_body(i, st2):
                c0, c1 = st2
                sl = pl.ds(si * GK + i * 16, 16)
                ed = edst[sl]
                es = esrc[sl]
                c0 = append(ps0, pd0, c0, es, ed,
                            (ed >= base0) & (ed < base0 + RPT), base0)
                c1 = append(ps1, pd1, c1, es, ed,
                            (ed >= base1) & (ed < base1 + RPT), base1)
                return c0, c1
            c0, c1 = lax.fori_loop(0, GK // 16, scan_body, (c0, c1))
            c0, w0 = flush(ps0, pd0, 0, c0, w0)
            c1, w1 = flush(ps1, pd1, 1, c1, w1)
            return (c0, w0, c1, w1)
        return lax.fori_loop(0, SUBS, sub_body, st)

    stage_start(0, 0)

    def pair_body(cp, st):
        for par in range(2):
            c = cp * 2 + par

            @pl.when(c + 1 < NCHUNK)
            def _(c=c, par=par):
                stage_start(c + 1, 1 - par)
            stage_wait(c, par)
            st = scan_chunk(par, st)
        return st
    st = lax.fori_loop(0, NCHUNK // 2, pair_body, (0, 0, 0, 0))
    if NCHUNK % 2:
        stage_wait(NCHUNK - 1, 0)
        st = scan_chunk(0, st)
    c0, w0, c1, w1 = st

    t0 = pl.multiple_of(w * EPF + w0, FB)
    t1 = pl.multiple_of((NTILES + w) * EPF + w1, FB)
    pltpu.sync_copy(ps0.at[pl.ds(0, FB)], lsrc_hbm.at[pl.ds(t0, FB)])
    pltpu.sync_copy(pd0.at[pl.ds(0, FB)], ldl_hbm.at[pl.ds(t0, FB)])
    pltpu.sync_copy(ps1.at[pl.ds(0, FB)], lsrc_hbm.at[pl.ds(t1, FB)])
    pltpu.sync_copy(pd1.at[pl.ds(0, FB)], ldl_hbm.at[pl.ds(t1, FB)])
    cntb[...] = jnp.full((16,), 0, jnp.int32) + (w0 + c0)
    pltpu.sync_copy(cntb, lcnt_hbm.at[pl.ds(pl.multiple_of(w * 16, 16), 16)])
    cntb[...] = jnp.full((16,), 0, jnp.int32) + (w1 + c1)
    pltpu.sync_copy(cntb, lcnt_hbm.at[pl.ds(pl.multiple_of((NTILES + w) * 16, 16), 16)])


def _make_agg(node_base):
  H = node_base // NPH

  @functools.partial(
    pl.kernel,
    out_type=(
        jax.ShapeDtypeStruct((NPH, D), jnp.float32),
        jax.ShapeDtypeStruct((NPH, D), jnp.float32),
        jax.ShapeDtypeStruct((NPH, D), jnp.float32),
        jax.ShapeDtypeStruct((NPH, 16), jnp.float32),
    ),
    mesh=_mesh,
    scratch_types=[
        pltpu.VMEM((GK,), jnp.int32),
        pltpu.VMEM((GK,), jnp.int32),
        pltpu.VMEM((GK,), jnp.int32),
        pltpu.VMEM((GK,), jnp.int32),
        pltpu.VMEM((GK,), jnp.int32),
        pltpu.VMEM((GK,), jnp.int32),
        pltpu.VMEM((16,), jnp.int32),
        pltpu.VMEM((GK, D), jnp.float32),
        pltpu.VMEM((GK, D), jnp.float32),
        pltpu.VMEM((ACCR, D), jnp.float32),
        pltpu.VMEM((ACCR, D), jnp.float32),
        pltpu.VMEM((ACCR, 16), jnp.float32),
        pltpu.VMEM_SHARED((SHROWS, D), jnp.float32),
        pltpu.SemaphoreType.DMA,
        pltpu.SemaphoreType.DMA,
    ],
    compiler_params=pltpu.CompilerParams(needs_layout_passes=False),
  )
  def _agg(hf_hbm, lsrc_hbm, ldl_hbm, lcnt_hbm,
           ssum_hbm, smx_hbm, smn_hbm, sdeg_hbm,
           cgath, cdlg, cdl2, cgath2, cdlg2, cdl22, cntb,
           rows, rows2, amx, amn, adeg, sh, sem, sem2):
    cc = lax.axis_index("c")
    wl = lax.axis_index("s")
    w = wl * 2 + cc
    obase = w * RPT
    shbase = wl * RPT

    zf = jnp.zeros((16,), jnp.float32)
    ninf = jnp.full((16,), -jnp.inf, jnp.float32)
    pinf = jnp.full((16,), jnp.inf, jnp.float32)
    one16 = jnp.full((16,), 1.0, jnp.float32)
    lane = lax.iota(jnp.int32, 16)

    def zrows(i, _):
        for f in range(D // 16):
            rows[i, pl.ds(f * 16, 16)] = zf
        return 0
    lax.fori_loop(0, GK, zrows, 0)

    def zacc(i, _):
        for f in range(D // 16):
            amx[i, pl.ds(f * 16, 16)] = ninf
            amn[i, pl.ds(f * 16, 16)] = pinf
        adeg[i, :] = zf
        return 0
    lax.fori_loop(0, ACCR, zacc, 0)

    pltpu.sync_copy(rows, sh.at[pl.ds(shbase, GK)])
    pltpu.sync_copy(rows.at[pl.ds(0, RPT - GK)], sh.at[pl.ds(shbase + GK, RPT - GK)])

    @pl.when(wl == 0)
    def _():
        pltpu.sync_copy(rows, sh.at[pl.ds(DUMMY_SH, GK)])

    pltpu.sync_copy(lcnt_hbm.at[pl.ds(pl.multiple_of((H * NTILES + w) * 16, 16), 16)], cntb)
    cnt = jnp.max(cntb[...])
    rbase = (H * NTILES + w) * EPF

    bufs = ((cgath, cdlg, cdl2, rows, sem), (cgath2, cdlg2, cdl22, rows2, sem2))
    nf = D // 16

    def load_group(gi, b):
        cg, cd, c2, rw, sm = bufs[b]
        gof = pl.multiple_of(rbase + gi * GK, GK)
        pltpu.sync_copy(lsrc_hbm.at[pl.ds(gof, GK)], cg)
        pltpu.sync_copy(ldl_hbm.at[pl.ds(gof, GK)], cd)
        for k in range(GK // 16):
            d1 = pl.ds(k * 16, 16)
            valid = (gi * GK + k * 16 + lane) < cnt
            sv = jnp.where(valid, cg[d1], 0)
            dv = jnp.where(valid, cd[d1], RPT)
            cg[d1] = sv
            cd[d1] = dv
            c2[d1] = jnp.where(dv >= RPT, DUMMY_SH, dv + shbase)
        pltpu.async_copy(hf_hbm.at[cg], rw, sm)

    def process_group(b):
        cg, cd, c2, rw, sm = bufs[b]
        pltpu.make_async_copy(hf_hbm.at[cg], rw, sm).wait()
        pltpu.sync_copy(rw, sh.at[c2], add=True)

        def kbody(k, _):
            eb = k * 16
            dlv = cd[pl.ds(eb, 16)]
            for j in range(16):
                ej = eb + j
                dlb = dlv[jnp.full((16,), j, jnp.int32)]
                plsc.addupdate_scatter(adeg, [dlb, lane], one16)
                cols = [lane + (f * 16) for f in range(nf)]
                rs = [rw[ej, pl.ds(f * 16, 16)] for f in range(nf)]
                mxs = [plsc.load_gather(amx, [dlb, cols[f]]) for f in range(nf)]
                for f in range(nf):
                    plsc.store_scatter(amx, [dlb, cols[f]],
                                       jnp.maximum(mxs[f], rs[f]))
                mns = [plsc.load_gather(amn, [dlb, cols[f]]) for f in range(nf)]
                for f in range(nf):
                    plsc.store_scatter(amn, [dlb, cols[f]],
                                       jnp.minimum(mns[f], rs[f]))
            return 0
        lax.fori_loop(0, GK // 16, kbody, 0)

    @pl.when(0 < cnt)
    def _():
        load_group(0, 0)

    def gouter(go, _):
        for par in range(2):
            gi2 = go * 2 + par

            @pl.when(gi2 * GK < cnt)
            def _(gi2=gi2, par=par):
                @pl.when((gi2 + 1) * GK < cnt)
                def _():
                    load_group(gi2 + 1, 1 - par)
                process_group(par)
        return 0
    lax.fori_loop(0, NGMAX // 2 + 1, gouter, 0)

    pltpu.sync_copy(amx.at[pl.ds(0, RPT)], smx_hbm.at[pl.ds(obase, RPT)])
    pltpu.sync_copy(amn.at[pl.ds(0, RPT)], smn_hbm.at[pl.ds(obase, RPT)])
    pltpu.sync_copy(adeg.at[pl.ds(0, RPT)], sdeg_hbm.at[pl.ds(obase, RPT)])
    pltpu.sync_copy(sh.at[pl.ds(shbase, RPT)], ssum_hbm.at[pl.ds(obase, RPT)])
  return _agg


_agg_lo = _make_agg(0)
_agg_hi = _make_agg(NPH)


def _c1_body(ssum, smx, smn, sdeg, snorm, W, b, hn, stats):
    deg = sdeg[...][:, 0:1]
    degc = jnp.maximum(deg, 1.0)
    mean = ssum[...] / degc
    has = deg > 0.0
    mx = jnp.where(has, smx[...], 0.0)
    mn = jnp.where(has, smn[...], 0.0)
    logd = jnp.log(deg + 1.0)
    amp = logd * (1.0 / AVG_D_LOG)
    att = AVG_D_LOG / jnp.maximum(logd, 1e-6)
    agg = jnp.concatenate([mean, mx, mn], axis=1)
    h1 = (jnp.dot(agg, W[0:3 * D, :], preferred_element_type=jnp.float32)
          + jnp.dot(agg * amp, W[3 * D:6 * D, :], preferred_element_type=jnp.float32)
          + jnp.dot(agg * att, W[6 * D:9 * D, :], preferred_element_type=jnp.float32)
          + b[...])
    h1 = h1 * snorm[...]
    hn[...] = h1
    i = pl.program_id(0)

    @pl.when(i == 0)
    def _():
        stats[...] = jnp.zeros_like(stats)

    rid = i * BN + lax.broadcasted_iota(jnp.int32, (BN, 1), 0)
    valid = rid < N
    hv = jnp.where(valid, h1, 0.0)
    hv2 = jnp.where(valid, h1 * h1, 0.0)
    stats[0:1, :] = stats[0:1, :] + jnp.sum(hv, axis=0, keepdims=True)
    stats[1:2, :] = stats[1:2, :] + jnp.sum(hv2, axis=0, keepdims=True)


_c1 = pl.pallas_call(
    _c1_body,
    grid=(NB,),
    in_specs=[
        pl.BlockSpec((BN, D), lambda i: (i, 0)),
        pl.BlockSpec((BN, D), lambda i: (i, 0)),
        pl.BlockSpec((BN, D), lambda i: (i, 0)),
        pl.BlockSpec((BN, 16), lambda i: (i, 0)),
        pl.BlockSpec((BN, 1), lambda i: (i, 0)),
        pl.BlockSpec((9 * D, D), lambda i: (0, 0)),
        pl.BlockSpec((1, D), lambda i: (0, 0)),
    ],
    out_specs=[
        pl.BlockSpec((BN, D), lambda i: (i, 0)),
        pl.BlockSpec((8, D), lambda i: (0, 0)),
    ],
    out_shape=[
        jax.ShapeDtypeStruct((NP, D), jnp.float32),
        jax.ShapeDtypeStruct((8, D), jnp.float32),
    ],
)


def _c2_body(hn, hf, stats, gamma, beta, out):
    mu = stats[0:1, :] * (1.0 / N)
    ex2 = stats[1:2, :] * (1.0 / N)
    var = ex2 - mu * mu
    inv = lax.rsqrt(var + 1e-5)
    out[...] = hf[...] + jnp.maximum((hn[...] - mu) * inv * gamma[...] + beta[...], 0.0)


_c2 = pl.pallas_call(
    _c2_body,
    grid=(NB,),
    in_specs=[
        pl.BlockSpec((BN, D), lambda i: (i, 0)),
        pl.BlockSpec((BN, D), lambda i: (i, 0)),
        pl.BlockSpec((8, D), lambda i: (0, 0)),
        pl.BlockSpec((1, D), lambda i: (0, 0)),
        pl.BlockSpec((1, D), lambda i: (0, 0)),
    ],
    out_specs=pl.BlockSpec((BN, D), lambda i: (i, 0)),
    out_shape=jax.ShapeDtypeStruct((NP, D), jnp.float32),
)


def _ro_body(hf, w0, b0, w1, b1, w2, b2, out):
    z = jnp.maximum(jnp.dot(hf[...], w0[...], preferred_element_type=jnp.float32) + b0[...], 0.0)
    z = jnp.maximum(jnp.dot(z, w1[...], preferred_element_type=jnp.float32) + b1[...], 0.0)
    out[...] = jnp.dot(z, w2[...], preferred_element_type=jnp.float32) + b2[...]


_ro = pl.pallas_call(
    _ro_body,
    grid=(NB,),
    in_specs=[
        pl.BlockSpec((BN, D), lambda i: (i, 0)),
        pl.BlockSpec((D, D // 2), lambda i: (0, 0)),
        pl.BlockSpec((1, D // 2), lambda i: (0, 0)),
        pl.BlockSpec((D // 2, D // 4), lambda i: (0, 0)),
        pl.BlockSpec((1, D // 4), lambda i: (0, 0)),
        pl.BlockSpec((D // 4, NCLS), lambda i: (0, 0)),
        pl.BlockSpec((1, NCLS), lambda i: (0, 0)),
    ],
    out_specs=pl.BlockSpec((BN, NCLS), lambda i: (i, 0)),
    out_shape=jax.ShapeDtypeStruct((NP, NCLS), jnp.float32),
)


def kernel(g, h, e, snorm_n, snorm_e, emb,
           W0, b0, gamma0, beta0,
           W1, b1, gamma1, beta1,
           W2, b2, gamma2, beta2,
           W3, b3, gamma3, beta3,
           Wr0, br0, Wr1, br1, Wr2, br2):
    src, dst = g[0], g[1]
    hp = jnp.concatenate([h, jnp.zeros((NP - N,), jnp.int32)])
    srcp = jnp.concatenate([src, jnp.zeros((EP - E,), jnp.int32)])
    dstp = jnp.concatenate([dst, jnp.full((EP - E,), 1 << 20, jnp.int32)])
    snp = jnp.concatenate([snorm_n, jnp.ones((NP - N, 1), jnp.float32)], axis=0)

    hf = _embed(emb, hp)
    lsrc, ldl, lcnt = _part(srcp, dstp)
    for (W, b, ga, be) in ((W0, b0, gamma0, beta0), (W1, b1, gamma1, beta1),
                           (W2, b2, gamma2, beta2), (W3, b3, gamma3, beta3)):
        s_lo, mx_lo, mn_lo, dg_lo = _agg_lo(hf, lsrc, ldl, lcnt)
        s_hi, mx_hi, mn_hi, dg_hi = _agg_hi(hf, lsrc, ldl, lcnt)
        ssum = jnp.concatenate([s_lo, s_hi], axis=0)
        smx = jnp.concatenate([mx_lo, mx_hi], axis=0)
        smn = jnp.concatenate([mn_lo, mn_hi], axis=0)
        sdeg = jnp.concatenate([dg_lo, dg_hi], axis=0)
        hn, stats = _c1(ssum, smx, smn, sdeg, snp, W, b.reshape(1, D))
        hf = _c2(hn, hf, stats, ga.reshape(1, D), be.reshape(1, D))
    z = _ro(hf, Wr0, br0.reshape(1, -1), Wr1, br1.reshape(1, -1),
            Wr2, br2.reshape(1, -1))
    return z[:N]


# speedup vs baseline: 4.6467x; 1.1217x over previous
import functools

import jax
import jax.numpy as jnp
from jax import lax
from jax.experimental import pallas as pl
from jax.experimental.pallas import tpu as pltpu
from jax.experimental.pallas import tpu_sc as plsc
import numpy as np

N = 10000
NP = 10240
E = 320000
D = 128
NCLS = 8
AVG_D_LOG = float(np.log(32.0))

NTILES = 32
NPH = NP // 2
RPT = NPH // NTILES
ERPT = NP // NTILES
CH = 2048
GK = 128
SUBS = CH // GK
EP = ((E + CH - 1) // CH) * CH
NCHUNK = EP // CH
CB = 2 * GK + 32
FB = 2048
PB = FB + GK + 32
EPF = EP + FB
NGMAX = EPF // GK
ACCR = RPT + 8
SHROWS = 16 * RPT + GK
DUMMY_SH = 16 * RPT
BN = 1024
NB = NP // BN

_mesh = plsc.VectorSubcoreMesh(core_axis_name="c", subcore_axis_name="s")


@functools.partial(
    pl.kernel,
    out_type=jax.ShapeDtypeStruct((NP, D), jnp.float32),
    mesh=_mesh,
    scratch_types=[
        pltpu.VMEM((64,), jnp.int32),
        pltpu.VMEM((64, D), jnp.float32),
        pltpu.SemaphoreType.DMA,
    ],
    compiler_params=pltpu.CompilerParams(needs_layout_passes=False),
)
def _embed(emb_hbm, h_hbm, out_hbm, idx_v, rows_v, sem):
    w = lax.axis_index("s") * 2 + lax.axis_index("c")
    base = w * ERPT
    for g in range(ERPT // 64):
        pltpu.sync_copy(h_hbm.at[pl.ds(base + g * 64, 64)], idx_v)
        pltpu.async_copy(emb_hbm.at[idx_v], rows_v, sem).wait()
        pltpu.sync_copy(rows_v, out_hbm.at[pl.ds(base + g * 64, 64)])


@functools.partial(
    pl.kernel,
    out_type=(
        jax.ShapeDtypeStruct((2 * NTILES * EPF,), jnp.int32),
        jax.ShapeDtypeStruct((2 * NTILES * EPF,), jnp.int32),
        jax.ShapeDtypeStruct((2 * NTILES * 16,), jnp.int32),
    ),
    mesh=_mesh,
    scratch_types=[
        pltpu.VMEM((PB,), jnp.int32),
        pltpu.VMEM((PB,), jnp.int32),
        pltpu.VMEM((PB,), jnp.int32),
        pltpu.VMEM((PB,), jnp.int32),
        pltpu.VMEM((CH,), jnp.int32),
        pltpu.VMEM((CH,), jnp.int32),
        pltpu.VMEM((CH,), jnp.int32),
        pltpu.VMEM((CH,), jnp.int32),
        pltpu.VMEM((16,), jnp.int32),
        pltpu.SemaphoreType.DMA,
        pltpu.SemaphoreType.DMA,
    ],
    compiler_params=pltpu.CompilerParams(needs_layout_passes=False),
)
def _part(src_hbm, dst_hbm, lsrc_hbm, ldl_hbm, lcnt_hbm,
          ps0, pd0, ps1, pd1, esrcA, edstA, esrcB, edstB, cntb, semA, semB):
    cc = lax.axis_index("c")
    wl = lax.axis_index("s")
    w = wl * 2 + cc
    base0 = w * RPT
    base1 = NPH + w * RPT
    lane = lax.iota(jnp.int32, 16)

    def append(psrc, pdl, cnt, es, ed, m, bs):
        mi = m.astype(jnp.int32)
        pos = plsc.cumsum(mi) - 1
        tgt = jnp.where(m, cnt + pos, PB - 16 + lane)
        plsc.store_scatter(psrc, [tgt], es)
        plsc.store_scatter(pdl, [tgt], ed - bs)
        return cnt + jnp.sum(mi)

    def flush(psrc, pdl, h, cnt, wr):
        full = cnt >= FB
        rbase = pl.multiple_of((h * NTILES + w) * EPF + wr, FB)

        @pl.when(full)
        def _():
            pltpu.sync_copy(psrc.at[pl.ds(0, FB)], lsrc_hbm.at[pl.ds(rbase, FB)])
            pltpu.sync_copy(pdl.at[pl.ds(0, FB)], ldl_hbm.at[pl.ds(rbase, FB)])
            for j in range(GK // 16):
                ssl = pl.ds(FB + j * 16, 16)
                v1 = psrc[ssl]
                v2 = pdl[ssl]
                psrc[pl.ds(j * 16, 16)] = v1
                pdl[pl.ds(j * 16, 16)] = v2
        return (jnp.where(full, cnt - FB, cnt), jnp.where(full, wr + FB, wr))

    stg = ((esrcA, edstA, semA), (esrcB, edstB, semB))

    def stage_start(c, b):
        es, ed, sm = stg[b]
        pltpu.async_copy(src_hbm.at[pl.ds(c * CH, CH)], es, sm)
        pltpu.async_copy(dst_hbm.at[pl.ds(c * CH, CH)], ed, sm)

    def stage_wait(c, b):
        es, ed, sm = stg[b]
        pltpu.make_async_copy(src_hbm.at[pl.ds(c * CH, CH)], es, sm).wait()
        pltpu.make_async_copy(dst_hbm.at[pl.ds(c * CH, CH)], ed, sm).wait()

    def scan_chunk(b, st):
        esrc, edst = stg[b][0], stg[b][1]

        def sub_body(si, st):
            c0, w0, c1, w1 = st

            def scan_body(i, st2):
                c0, c1 = st2
                sl = pl.ds(si * GK + i * 16, 16)
                ed = edst[sl]
                es = esrc[sl]
                c0 = append(ps0, pd0, c0, es, ed,
                            (ed >= base0) & (ed < base0 + RPT), base0)
                c1 = append(ps1, pd1, c1, es, ed,
                            (ed >= base1) & (ed < base1 + RPT), base1)
                return c0, c1
            c0, c1 = lax.fori_loop(0, GK // 16, scan_body, (c0, c1))
            c0, w0 = flush(ps0, pd0, 0, c0, w0)
            c1, w1 = flush(ps1, pd1, 1, c1, w1)
            return (c0, w0, c1, w1)
        return lax.fori_loop(0, SUBS, sub_body, st)

    stage_start(0, 0)

    def pair_body(cp, st):
        for par in range(2):
            c = cp * 2 + par

            @pl.when(c + 1 < NCHUNK)
            def _(c=c, par=par):
                stage_start(c + 1, 1 - par)
            stage_wait(c, par)
            st = scan_chunk(par, st)
        return st
    st = lax.fori_loop(0, NCHUNK // 2, pair_body, (0, 0, 0, 0))
    if NCHUNK % 2:
        stage_wait(NCHUNK - 1, 0)
        st = scan_chunk(0, st)
    c0, w0, c1, w1 = st

    t0 = pl.multiple_of(w * EPF + w0, FB)
    t1 = pl.multiple_of((NTILES + w) * EPF + w1, FB)
    pltpu.sync_copy(ps0.at[pl.ds(0, FB)], lsrc_hbm.at[pl.ds(t0, FB)])
    pltpu.sync_copy(pd0.at[pl.ds(0, FB)], ldl_hbm.at[pl.ds(t0, FB)])
    pltpu.sync_copy(ps1.at[pl.ds(0, FB)], lsrc_hbm.at[pl.ds(t1, FB)])
    pltpu.sync_copy(pd1.at[pl.ds(0, FB)], ldl_hbm.at[pl.ds(t1, FB)])
    cntb[...] = jnp.full((16,), 0, jnp.int32) + (w0 + c0)
    pltpu.sync_copy(cntb, lcnt_hbm.at[pl.ds(pl.multiple_of(w * 16, 16), 16)])
    cntb[...] = jnp.full((16,), 0, jnp.int32) + (w1 + c1)
    pltpu.sync_copy(cntb, lcnt_hbm.at[pl.ds(pl.multiple_of((NTILES + w) * 16, 16), 16)])


def _make_agg(node_base):
  H = node_base // NPH

  @functools.partial(
    pl.kernel,
    out_type=(
        jax.ShapeDtypeStruct((NPH, D), jnp.float32),
        jax.ShapeDtypeStruct((NPH, D), jnp.float32),
        jax.ShapeDtypeStruct((NPH, D), jnp.float32),
        jax.ShapeDtypeStruct((NPH, 16), jnp.float32),
    ),
    mesh=_mesh,
    scratch_types=[
        pltpu.VMEM((GK,), jnp.int32),
        pltpu.VMEM((GK,), jnp.int32),
        pltpu.VMEM((GK,), jnp.int32),
        pltpu.VMEM((GK,), jnp.int32),
        pltpu.VMEM((GK,), jnp.int32),
        pltpu.VMEM((GK,), jnp.int32),
        pltpu.VMEM((16,), jnp.int32),
        pltpu.VMEM((GK, D), jnp.float32),
        pltpu.VMEM((GK, D), jnp.float32),
        pltpu.VMEM((ACCR, D), jnp.float32),
        pltpu.VMEM((ACCR, D), jnp.float32),
        pltpu.VMEM((ACCR, 16), jnp.float32),
        pltpu.VMEM_SHARED((SHROWS, D), jnp.float32),
        pltpu.SemaphoreType.DMA,
        pltpu.SemaphoreType.DMA,
        pltpu.SemaphoreType.DMA,
        pltpu.SemaphoreType.DMA,
    ],
    compiler_params=pltpu.CompilerParams(needs_layout_passes=False),
  )
  def _agg(hf_hbm, lsrc_hbm, ldl_hbm, lcnt_hbm,
           ssum_hbm, smx_hbm, smn_hbm, sdeg_hbm,
           cgath, cdlg, cdl2, cgath2, cdlg2, cdl22, cntb,
           rows, rows2, amx, amn, adeg, sh, sem, sem2, semi, sems):
    cc = lax.axis_index("c")
    wl = lax.axis_index("s")
    w = wl * 2 + cc
    obase = w * RPT
    shbase = wl * RPT

    zf = jnp.zeros((16,), jnp.float32)
    ninf = jnp.full((16,), -jnp.inf, jnp.float32)
    pinf = jnp.full((16,), jnp.inf, jnp.float32)
    one16 = jnp.full((16,), 1.0, jnp.float32)
    lane = lax.iota(jnp.int32, 16)

    def zrows(i, _):
        for f in range(D // 16):
            rows[i, pl.ds(f * 16, 16)] = zf
        return 0
    lax.fori_loop(0, GK, zrows, 0)

    def zacc(i, _):
        for f in range(D // 16):
            amx[i, pl.ds(f * 16, 16)] = ninf
            amn[i, pl.ds(f * 16, 16)] = pinf
        adeg[i, :] = zf
        return 0
    lax.fori_loop(0, ACCR, zacc, 0)

    pltpu.sync_copy(rows, sh.at[pl.ds(shbase, GK)])
    pltpu.sync_copy(rows.at[pl.ds(0, RPT - GK)], sh.at[pl.ds(shbase + GK, RPT - GK)])

    @pl.when(wl == 0)
    def _():
        pltpu.sync_copy(rows, sh.at[pl.ds(DUMMY_SH, GK)])

    pltpu.sync_copy(lcnt_hbm.at[pl.ds(pl.multiple_of((H * NTILES + w) * 16, 16), 16)], cntb)
    cnt = jnp.max(cntb[...])
    rbase = (H * NTILES + w) * EPF

    bufs = ((cgath, cdlg, cdl2, rows, sem), (cgath2, cdlg2, cdl22, rows2, sem2))
    nf = D // 16

    def load_group(gi, b):
        cg, cd, c2, rw, sm = bufs[b]
        gof = pl.multiple_of(rbase + gi * GK, GK)
        pltpu.async_copy(lsrc_hbm.at[pl.ds(gof, GK)], cg, semi)
        pltpu.async_copy(ldl_hbm.at[pl.ds(gof, GK)], cd, semi)
        pltpu.make_async_copy(lsrc_hbm.at[pl.ds(gof, GK)], cg, semi).wait()
        pltpu.make_async_copy(ldl_hbm.at[pl.ds(gof, GK)], cd, semi).wait()
        for k in range(GK // 16):
            d1 = pl.ds(k * 16, 16)
            valid = (gi * GK + k * 16 + lane) < cnt
            sv = jnp.where(valid, cg[d1], 0)
            dv = jnp.where(valid, cd[d1], RPT)
            cg[d1] = sv
            cd[d1] = dv
            c2[d1] = jnp.where(dv >= RPT, DUMMY_SH, dv + shbase)
        pltpu.async_copy(hf_hbm.at[cg], rw, sm)

    def process_group(b):
        cg, cd, c2, rw, sm = bufs[b]
        pltpu.make_async_copy(hf_hbm.at[cg], rw, sm).wait()
        pltpu.async_copy(rw, sh.at[c2], sems, add=True)

        def kbody(k, _):
            eb = k * 16
            dlv = cd[pl.ds(eb, 16)]
            for j in range(16):
                ej = eb + j
                dlb = dlv[jnp.full((16,), j, jnp.int32)]
                plsc.addupdate_scatter(adeg, [dlb, lane], one16)
                cols = [lane + (f * 16) for f in range(nf)]
                rs = [rw[ej, pl.ds(f * 16, 16)] for f in range(nf)]
                mxs = [plsc.load_gather(amx, [dlb, cols[f]]) for f in range(nf)]
                for f in range(nf):
                    plsc.store_scatter(amx, [dlb, cols[f]],
                                       jnp.maximum(mxs[f], rs[f]))
                mns = [plsc.load_gather(amn, [dlb, cols[f]]) for f in range(nf)]
                for f in range(nf):
                    plsc.store_scatter(amn, [dlb, cols[f]],
                                       jnp.minimum(mns[f], rs[f]))
            return 0
        lax.fori_loop(0, GK // 16, kbody, 0)
        pltpu.make_async_copy(rw, sh.at[c2], sems).wait()

    @pl.when(0 < cnt)
    def _():
        load_group(0, 0)

    def gouter(go, _):
        for par in range(2):
            gi2 = go * 2 + par

            @pl.when(gi2 * GK < cnt)
            def _(gi2=gi2, par=par):
                @pl.when((gi2 + 1) * GK < cnt)
                def _():
                    load_group(gi2 + 1, 1 - par)
                process_group(par)
        return 0
    lax.fori_loop(0, NGMAX // 2 + 1, gouter, 0)

    pltpu.sync_copy(amx.at[pl.ds(0, RPT)], smx_hbm.at[pl.ds(obase, RPT)])
    pltpu.sync_copy(amn.at[pl.ds(0, RPT)], smn_hbm.at[pl.ds(obase, RPT)])
    pltpu.sync_copy(adeg.at[pl.ds(0, RPT)], sdeg_hbm.at[pl.ds(obase, RPT)])
    pltpu.sync_copy(sh.at[pl.ds(shbase, RPT)], ssum_hbm.at[pl.ds(obase, RPT)])
  return _agg


_agg_lo = _make_agg(0)
_agg_hi = _make_agg(NPH)


def _c1_body(ssum, smx, smn, sdeg, snorm, W, b, hn, stats):
    deg = sdeg[...][:, 0:1]
    degc = jnp.maximum(deg, 1.0)
    mean = ssum[...] / degc
    has = deg > 0.0
    mx = jnp.where(has, smx[...], 0.0)
    mn = jnp.where(has, smn[...], 0.0)
    logd = jnp.log(deg + 1.0)
    amp = logd * (1.0 / AVG_D_LOG)
    att = AVG_D_LOG / jnp.maximum(logd, 1e-6)
    agg = jnp.concatenate([mean, mx, mn], axis=1)
    h1 = (jnp.dot(agg, W[0:3 * D, :], preferred_element_type=jnp.float32)
          + jnp.dot(agg * amp, W[3 * D:6 * D, :], preferred_element_type=jnp.float32)
          + jnp.dot(agg * att, W[6 * D:9 * D, :], preferred_element_type=jnp.float32)
          + b[...])
    h1 = h1 * snorm[...]
    hn[...] = h1
    i = pl.program_id(0)

    @pl.when(i == 0)
    def _():
        stats[...] = jnp.zeros_like(stats)

    rid = i * BN + lax.broadcasted_iota(jnp.int32, (BN, 1), 0)
    valid = rid < N
    hv = jnp.where(valid, h1, 0.0)
    hv2 = jnp.where(valid, h1 * h1, 0.0)
    stats[0:1, :] = stats[0:1, :] + jnp.sum(hv, axis=0, keepdims=True)
    stats[1:2, :] = stats[1:2, :] + jnp.sum(hv2, axis=0, keepdims=True)


_c1 = pl.pallas_call(
    _c1_body,
    grid=(NB,),
    in_specs=[
        pl.BlockSpec((BN, D), lambda i: (i, 0)),
        pl.BlockSpec((BN, D), lambda i: (i, 0)),
        pl.BlockSpec((BN, D), lambda i: (i, 0)),
        pl.BlockSpec((BN, 16), lambda i: (i, 0)),
        pl.BlockSpec((BN, 1), lambda i: (i, 0)),
        pl.BlockSpec((9 * D, D), lambda i: (0, 0)),
        pl.BlockSpec((1, D), lambda i: (0, 0)),
    ],
    out_specs=[
        pl.BlockSpec((BN, D), lambda i: (i, 0)),
        pl.BlockSpec((8, D), lambda i: (0, 0)),
    ],
    out_shape=[
        jax.ShapeDtypeStruct((NP, D), jnp.float32),
        jax.ShapeDtypeStruct((8, D), jnp.float32),
    ],
)


def _c2_body(hn, hf, stats, gamma, beta, out):
    mu = stats[0:1, :] * (1.0 / N)
    ex2 = stats[1:2, :] * (1.0 / N)
    var = ex2 - mu * mu
    inv = lax.rsqrt(var + 1e-5)
    out[...] = hf[...] + jnp.maximum((hn[...] - mu) * inv * gamma[...] + beta[...], 0.0)


_c2 = pl.pallas_call(
    _c2_body,
    grid=(NB,),
    in_specs=[
        pl.BlockSpec((BN, D), lambda i: (i, 0)),
        pl.BlockSpec((BN, D), lambda i: (i, 0)),
        pl.BlockSpec((8, D), lambda i: (0, 0)),
        pl.BlockSpec((1, D), lambda i: (0, 0)),
        pl.BlockSpec((1, D), lambda i: (0, 0)),
    ],
    out_specs=pl.BlockSpec((BN, D), lambda i: (i, 0)),
    out_shape=jax.ShapeDtypeStruct((NP, D), jnp.float32),
)


def _ro_body(hf, w0, b0, w1, b1, w2, b2, out):
    z = jnp.maximum(jnp.dot(hf[...], w0[...], preferred_element_type=jnp.float32) + b0[...], 0.0)
    z = jnp.maximum(jnp.dot(z, w1[...], preferred_element_type=jnp.float32) + b1[...], 0.0)
    out[...] = jnp.dot(z, w2[...], preferred_element_type=jnp.float32) + b2[...]


_ro = pl.pallas_call(
    _ro_body,
    grid=(NB,),
    in_specs=[
        pl.BlockSpec((BN, D), lambda i: (i, 0)),
        pl.BlockSpec((D, D // 2), lambda i: (0, 0)),
        pl.BlockSpec((1, D // 2), lambda i: (0, 0)),
        pl.BlockSpec((D // 2, D // 4), lambda i: (0, 0)),
        pl.BlockSpec((1, D // 4), lambda i: (0, 0)),
        pl.BlockSpec((D // 4, NCLS), lambda i: (0, 0)),
        pl.BlockSpec((1, NCLS), lambda i: (0, 0)),
    ],
    out_specs=pl.BlockSpec((BN, NCLS), lambda i: (i, 0)),
    out_shape=jax.ShapeDtypeStruct((NP, NCLS), jnp.float32),
)


def kernel(g, h, e, snorm_n, snorm_e, emb,
           W0, b0, gamma0, beta0,
           W1, b1, gamma1, beta1,
           W2, b2, gamma2, beta2,
           W3, b3, gamma3, beta3,
           Wr0, br0, Wr1, br1, Wr2, br2):
    src, dst = g[0], g[1]
    hp = jnp.concatenate([h, jnp.zeros((NP - N,), jnp.int32)])
    srcp = jnp.concatenate([src, jnp.zeros((EP - E,), jnp.int32)])
    dstp = jnp.concatenate([dst, jnp.full((EP - E,), 1 << 20, jnp.int32)])
    snp = jnp.concatenate([snorm_n, jnp.ones((NP - N, 1), jnp.float32)], axis=0)

    hf = _embed(emb, hp)
    lsrc, ldl, lcnt = _part(srcp, dstp)
    for (W, b, ga, be) in ((W0, b0, gamma0, beta0), (W1, b1, gamma1, beta1),
                           (W2, b2, gamma2, beta2), (W3, b3, gamma3, beta3)):
        s_lo, mx_lo, mn_lo, dg_lo = _agg_lo(hf, lsrc, ldl, lcnt)
        s_hi, mx_hi, mn_hi, dg_hi = _agg_hi(hf, lsrc, ldl, lcnt)
        ssum = jnp.concatenate([s_lo, s_hi], axis=0)
        smx = jnp.concatenate([mx_lo, mx_hi], axis=0)
        smn = jnp.concatenate([mn_lo, mn_hi], axis=0)
        sdeg = jnp.concatenate([dg_lo, dg_hi], axis=0)
        hn, stats = _c1(ssum, smx, smn, sdeg, snp, W, b.reshape(1, D))
        hf = _c2(hn, hf, stats, ga.reshape(1, D), be.reshape(1, D))
    z = _ro(hf, Wr0, br0.reshape(1, -1), Wr1, br1.reshape(1, -1),
            Wr2, br2.reshape(1, -1))
    return z[:N]
